# Initial kernel scaffold; baseline (speedup 1.0000x reference)
import jax
import jax.numpy as jnp
from jax.experimental import pallas as pl


def kernel(x, edge_index_loop, edge_index_dep, edge_index_rdep, W0_loop, b0_loop, W0_dep, b0_dep, W0_rdep, b0_rdep, W1_loop, b1_loop, W1_dep, b1_dep, W1_rdep, b1_rdep):
    raise NotImplementedError("write your pallas kernel here")


# speedup vs baseline: 2.0120x; 2.0120x over previous
import functools
import jax
import jax.numpy as jnp
from jax import lax
from jax.experimental import pallas as pl
from jax.experimental.pallas import tpu as pltpu, tpu_sc as plsc

N = 10000
D = 128
N_PAD = 10240
SINK = 10200
K = 128
NW = 32
ROWS_PER_TILE = N_PAD // 16
BLK = 1024
GRID = N_PAD // BLK

E_LOOP_PAD = 12288
E_DEP_PAD = 163840
CPW = {"loop": 3, "dep": 40, "rdep": 40}


def _pad_edges(ei, e_pad):
    e = ei.shape[1]
    pad = jnp.full((e_pad - e,), SINK, dtype=jnp.int32)
    src = jnp.concatenate([ei[0].astype(jnp.int32), pad])
    dst = jnp.concatenate([ei[1].astype(jnp.int32), pad])
    return src, dst


def _deg_kernel(sl, dl, sd, dd, sr, dr, out_hbm, idx_v, ones_v, tmp_v, deg_sh, sem):
    cid = lax.axis_index("c")
    sid = lax.axis_index("s")
    wid = cid * 16 + sid
    row0 = sid * ROWS_PER_TILE

    z16 = jnp.zeros((16,), jnp.float32)

    def zinit(i, _):
        for j in range(8):
            ones_v[i, pl.ds(j * 16, 16)] = z16
            tmp_v[i, pl.ds(j * 16, 16)] = z16
        return 0
    lax.fori_loop(0, K, zinit, 0, unroll=False)

    for kk in range(ROWS_PER_TILE // K):
        pltpu.sync_copy(tmp_v, deg_sh.at[pl.ds(row0 + kk * K, K)])
    plsc.subcore_barrier()

    streams = [(sl, CPW["loop"]), (dl, CPW["loop"]),
               (sd, CPW["dep"]), (dd, CPW["dep"]),
               (sr, CPW["rdep"]), (dr, CPW["rdep"])]
    for q, (arr, cpw) in enumerate(streams):
        eq = jnp.where(lax.iota(jnp.int32, 16) == q, 1.0, 0.0).astype(jnp.float32)

        def init_body(i, _):
            ones_v[i, pl.ds(0, 16)] = eq
            return 0
        lax.fori_loop(0, K, init_body, 0, unroll=False)

        span = cpw * K
        base = wid * span

        def chunk_body(j, _):
            pltpu.sync_copy(arr.at[pl.ds(base + j * K, K)], idx_v)
            pltpu.sync_copy(ones_v, deg_sh.at[idx_v], add=True)
            return 0
        lax.fori_loop(0, cpw, chunk_body, 0, unroll=False)
    plsc.subcore_barrier()

    for kk in range(ROWS_PER_TILE // K):
        r0 = row0 + kk * K
        pltpu.sync_copy(deg_sh.at[pl.ds(r0, K)], tmp_v)
        pltpu.sync_copy(tmp_v, out_hbm.at[cid, pl.ds(r0, K)])


def _run_deg(sl, dl, sd, dd, sr, dr):
    k = pl.kernel(
        _deg_kernel,
        out_type=jax.ShapeDtypeStruct((2, N_PAD, D), jnp.float32),
        mesh=plsc.VectorSubcoreMesh(core_axis_name="c", subcore_axis_name="s"),
        scratch_types=[
            pltpu.VMEM((K,), jnp.int32),
            pltpu.VMEM((K, D), jnp.float32),
            pltpu.VMEM((K, D), jnp.float32),
            pltpu.VMEM_SHARED((N_PAD, D), jnp.float32),
            pltpu.SemaphoreType.DMA,
        ],
    )
    return k(sl, dl, sd, dd, sr, dr)


def _agg_kernel(yl, yd, yr, sl, dl, sd, dd, sr, dr, out_hbm,
                idx_s, idx_d, rows_v, zero_v, acc_sh, sem):
    cid = lax.axis_index("c")
    sid = lax.axis_index("s")
    wid = cid * 16 + sid
    row0 = sid * ROWS_PER_TILE

    z16 = jnp.zeros((16,), jnp.float32)

    def zinit(i, _):
        for j in range(8):
            zero_v[i, pl.ds(j * 16, 16)] = z16
        return 0
    lax.fori_loop(0, K, zinit, 0, unroll=False)

    rels = [(yl, sl, dl, CPW["loop"]), (yd, sd, dd, CPW["dep"]),
            (yr, sr, dr, CPW["rdep"])]
    for r, (ytab, sarr, darr, cpw) in enumerate(rels):
        for kk in range(ROWS_PER_TILE // K):
            pltpu.sync_copy(zero_v, acc_sh.at[pl.ds(row0 + kk * K, K)])
        plsc.subcore_barrier()

        span = cpw * K
        base = wid * span

        def chunk_body(j, _):
            off = base + j * K
            pltpu.sync_copy(sarr.at[pl.ds(off, K)], idx_s)
            pltpu.sync_copy(darr.at[pl.ds(off, K)], idx_d)
            pltpu.async_copy(ytab.at[idx_s], rows_v, sem).wait()
            pltpu.sync_copy(rows_v, acc_sh.at[idx_d], add=True)
            return 0
        lax.fori_loop(0, cpw, chunk_body, 0, unroll=False)
        plsc.subcore_barrier()

        for kk in range(ROWS_PER_TILE // K):
            r0 = row0 + kk * K
            pltpu.sync_copy(acc_sh.at[pl.ds(r0, K)], rows_v)
            pltpu.sync_copy(rows_v, out_hbm.at[cid, r, pl.ds(r0, K)])
        plsc.subcore_barrier()


def _run_agg(yl, yd, yr, sl, dl, sd, dd, sr, dr):
    k = pl.kernel(
        _agg_kernel,
        out_type=jax.ShapeDtypeStruct((2, 3, N_PAD, D), jnp.float32),
        mesh=plsc.VectorSubcoreMesh(core_axis_name="c", subcore_axis_name="s"),
        scratch_types=[
            pltpu.VMEM((K,), jnp.int32),
            pltpu.VMEM((K,), jnp.int32),
            pltpu.VMEM((K, D), jnp.float32),
            pltpu.VMEM((K, D), jnp.float32),
            pltpu.VMEM_SHARED((N_PAD, D), jnp.float32),
            pltpu.SemaphoreType.DMA,
        ],
    )
    return k(yl, yd, yr, sl, dl, sd, dd, sr, dr)


def _scale_kernel(degp_ref, out_ref):
    p = degp_ref[...]
    deg = (p[0] + p[1])[:, 0:8]
    out_ref[...] = lax.rsqrt(jnp.maximum(deg, 1.0))


def _run_scale(degp):
    return pl.pallas_call(
        _scale_kernel,
        grid=(GRID,),
        in_specs=[pl.BlockSpec((2, BLK, D), lambda i: (0, i, 0))],
        out_specs=pl.BlockSpec((BLK, 8), lambda i: (i, 0)),
        out_shape=jax.ShapeDtypeStruct((N_PAD, 8), jnp.float32),
    )(degp)


def _proj0_kernel(x_ref, sc_ref, wl_ref, wd_ref, wr_ref, yl_ref, yd_ref, yr_ref):
    x = x_ref[...]
    s = sc_ref[...]
    for w_ref, y_ref, col in ((wl_ref, yl_ref, 0), (wd_ref, yd_ref, 2),
                              (wr_ref, yr_ref, 4)):
        xs = x * s[:, col][:, None]
        y_ref[...] = jnp.dot(xs, w_ref[...],
                             preferred_element_type=jnp.float32,
                             precision=lax.Precision.HIGHEST)


def _run_proj0(x_pad, scales, w0l, w0d, w0r):
    row_spec = pl.BlockSpec((BLK, D), lambda i: (i, 0))
    return pl.pallas_call(
        _proj0_kernel,
        grid=(GRID,),
        in_specs=[row_spec,
                  pl.BlockSpec((BLK, 8), lambda i: (i, 0)),
                  pl.BlockSpec((D, D), lambda i: (0, 0)),
                  pl.BlockSpec((D, D), lambda i: (0, 0)),
                  pl.BlockSpec((D, D), lambda i: (0, 0))],
        out_specs=[row_spec, row_spec, row_spec],
        out_shape=[jax.ShapeDtypeStruct((N_PAD, D), jnp.float32)] * 3,
    )(x_pad, scales, w0l, w0d, w0r)


def _combine_proj_kernel(p_ref, sc_ref, bsum_ref, wl_ref, wd_ref, wr_ref,
                         yl_ref, yd_ref, yr_ref):
    i = pl.program_id(0)
    s = sc_ref[...]
    p = p_ref[...]
    acc = (p[0, 0] + p[1, 0]) * s[:, 1][:, None]
    acc += (p[0, 1] + p[1, 1]) * s[:, 3][:, None]
    acc += (p[0, 2] + p[1, 2]) * s[:, 5][:, None]
    acc += bsum_ref[...]
    rows = i * BLK + lax.broadcasted_iota(jnp.int32, (BLK, 1), 0)
    h = jnp.where(rows < N, jnp.maximum(acc, 0.0), 0.0)
    for w_ref, y_ref, col in ((wl_ref, yl_ref, 0), (wd_ref, yd_ref, 2),
                              (wr_ref, yr_ref, 4)):
        hs = h * s[:, col][:, None]
        y_ref[...] = jnp.dot(hs, w_ref[...],
                             preferred_element_type=jnp.float32,
                             precision=lax.Precision.HIGHEST)


def _run_combine_proj(p, scales, bsum, w1l, w1d, w1r):
    row_spec = pl.BlockSpec((BLK, D), lambda i: (i, 0))
    return pl.pallas_call(
        _combine_proj_kernel,
        grid=(GRID,),
        in_specs=[pl.BlockSpec((2, 3, BLK, D), lambda i: (0, 0, i, 0)),
                  pl.BlockSpec((BLK, 8), lambda i: (i, 0)),
                  pl.BlockSpec((1, D), lambda i: (0, 0)),
                  pl.BlockSpec((D, D), lambda i: (0, 0)),
                  pl.BlockSpec((D, D), lambda i: (0, 0)),
                  pl.BlockSpec((D, D), lambda i: (0, 0))],
        out_specs=[row_spec, row_spec, row_spec],
        out_shape=[jax.ShapeDtypeStruct((N_PAD, D), jnp.float32)] * 3,
    )(p, scales, bsum, w1l, w1d, w1r)


def _readout_kernel(p_ref, sc_ref, bsum_ref, out_ref):
    i = pl.program_id(0)
    s = sc_ref[...]
    p = p_ref[...]
    acc = (p[0, 0] + p[1, 0]) * s[:, 1][:, None]
    acc += (p[0, 1] + p[1, 1]) * s[:, 3][:, None]
    acc += (p[0, 2] + p[1, 2]) * s[:, 5][:, None]
    acc += bsum_ref[...]
    rows = i * BLK + lax.broadcasted_iota(jnp.int32, (BLK, 1), 0)
    h = jnp.where(rows < N, jnp.maximum(acc, 0.0), 0.0)
    part = jnp.sum(h, axis=0, keepdims=True) * (1.0 / N)

    @pl.when(i == 0)
    def _():
        out_ref[...] = part

    @pl.when(i > 0)
    def _():
        out_ref[...] += part


def _run_readout(p, scales, bsum):
    return pl.pallas_call(
        _readout_kernel,
        grid=(GRID,),
        in_specs=[pl.BlockSpec((2, 3, BLK, D), lambda i: (0, 0, i, 0)),
                  pl.BlockSpec((BLK, 8), lambda i: (i, 0)),
                  pl.BlockSpec((1, D), lambda i: (0, 0))],
        out_specs=pl.BlockSpec((1, D), lambda i: (0, 0)),
        out_shape=jax.ShapeDtypeStruct((1, D), jnp.float32),
    )(p, scales, bsum)


def kernel(x, edge_index_loop, edge_index_dep, edge_index_rdep,
           W0_loop, b0_loop, W0_dep, b0_dep, W0_rdep, b0_rdep,
           W1_loop, b1_loop, W1_dep, b1_dep, W1_rdep, b1_rdep):
    sl, dl = _pad_edges(edge_index_loop, E_LOOP_PAD)
    sd, dd = _pad_edges(edge_index_dep, E_DEP_PAD)
    sr, dr = _pad_edges(edge_index_rdep, E_DEP_PAD)
    x_pad = jnp.pad(x, ((0, N_PAD - N), (0, 0)))

    degp = _run_deg(sl, dl, sd, dd, sr, dr)
    scales = _run_scale(degp)

    b0sum = (b0_loop + b0_dep + b0_rdep).reshape(1, D)
    b1sum = (b1_loop + b1_dep + b1_rdep).reshape(1, D)

    y0l, y0d, y0r = _run_proj0(x_pad, scales, W0_loop, W0_dep, W0_rdep)
    p0 = _run_agg(y0l, y0d, y0r, sl, dl, sd, dd, sr, dr)
    y1l, y1d, y1r = _run_combine_proj(p0, scales, b0sum, W1_loop, W1_dep, W1_rdep)
    p1 = _run_agg(y1l, y1d, y1r, sl, dl, sd, dd, sr, dr)
    return _run_readout(p1, scales, b1sum)
